# Initial kernel scaffold; baseline (speedup 1.0000x reference)
import jax
import jax.numpy as jnp
from jax.experimental import pallas as pl


def kernel(x, pos, edge_index, W_emb, b_emb, fc1_w, fc1_b, fc2_w, fc2_b):
    raise NotImplementedError("write your pallas kernel here")


# speedup vs baseline: 8.0166x; 8.0166x over previous
import functools

import numpy as np
import jax
import jax.numpy as jnp
from jax import lax
from jax.experimental import pallas as pl
from jax.experimental.pallas import tpu as pltpu
from jax.experimental.pallas import tpu_sc as plsc

N_NODES = 10000
D_IN = 128
EMBED = 20
EMB_NUMBER = 32
NUM_BASIS = 8
MAX_RADIUS = 6.0
N_EDGES = 320000

TABLE_ROWS = 10016
NB = 512
LMAX = 8.0
HSTEP = LMAX / NB
INV_H = NB / LMAX
SCALE = 1.0 / np.sqrt(N_EDGES / N_NODES)

NC, NS = 2, 16
NW = NC * NS
BLK = 128
BPC = 4
CHUNKS = 20
BLOCKS_PER_TILE = BPC * CHUNKS
E_PAD = NW * BLOCKS_PER_TILE * BLK
ACC_ROWS = 10240


def _tc_prepare(x_pad, posc, W_p, b_p, f1w, f1b, f2w, f2b, table_o, lut_o):
    h = jnp.dot(x_pad[...], W_p[...], preferred_element_type=jnp.float32) + b_p[...]
    rows = lax.broadcasted_iota(jnp.int32, (TABLE_ROWS, 1), 0)
    valid = (rows < N_NODES).astype(jnp.float32)
    table_o[...] = h * valid + posc[...]

    L = lax.broadcasted_iota(jnp.int32, (NB + 8, 1), 0).astype(jnp.float32) * HSTEP
    step = MAX_RADIUS / (NUM_BASIS + 1)
    vals = (lax.broadcasted_iota(jnp.int32, (1, NUM_BASIS), 1)
            .astype(jnp.float32) + 1.0) * step
    diff = (L - vals) * (1.0 / step)
    emb = jnp.exp(-diff * diff) * (float(np.sqrt(NUM_BASIS)) / 1.12)
    hid = jnp.dot(emb, f1w[...], preferred_element_type=jnp.float32,
                  precision=lax.Precision.HIGHEST) + f1b[...]
    hid = hid / (1.0 + jnp.exp(-hid))
    w = jnp.dot(hid, f2w[...], preferred_element_type=jnp.float32,
                precision=lax.Precision.HIGHEST) + f2b[...]
    wg = w * SCALE
    base = wg[0:NB]
    slope = wg[1:NB + 1] - base
    lut_o[...] = jnp.concatenate([base, slope], axis=1)


def _tc_combine(parts, out_o):
    out_o[...] = parts[0, 0:N_NODES, 0:EMBED] + parts[1, 0:N_NODES, 0:EMBED]


def _sc_body(table, pos16, srcb, dstb, luth, part,
             lut_v, srci, dsti, rowsA, posD, msg, zbuf, accum,
             sem_g, sem_s, sem_i):
    c = lax.axis_index("c")
    s = lax.axis_index("s")
    wid = s * NC + c
    HALF = BPC * BLK

    pltpu.sync_copy(luth, lut_v)

    z = jnp.zeros((16,), jnp.float32)
    for r in range(16):
        zbuf[r, pl.ds(0, 16)] = z
        zbuf[r, pl.ds(16, 16)] = z

    def mzero(j, carry):
        msg[j, pl.ds(0, 16)] = z
        msg[j, pl.ds(16, 16)] = z
        return carry

    lax.fori_loop(0, 2 * HALF, mzero, 0)

    def zloop(j, carry):
        pltpu.sync_copy(zbuf, accum.at[pl.ds(s * 640 + j * 16, 16)])
        return carry

    lax.fori_loop(0, 40, zloop, 0)
    plsc.subcore_barrier()

    iota16 = lax.iota(jnp.int32, 16)

    def compute_chunk(ro):
        @plsc.parallel_loop(0, BPC * 8, 1, unroll=2)
        def group_compute(g):
            lane = ro + g * 16 + iota16

            def gath(ref, col):
                return plsc.load_gather(
                    ref, [lane, jnp.full((16,), col, jnp.int32)])

            pd0 = gath(posD, 0)
            pd1 = gath(posD, 1)
            pd2 = gath(posD, 2)
            ps0 = gath(rowsA, EMBED + 0)
            ps1 = gath(rowsA, EMBED + 1)
            ps2 = gath(rowsA, EMBED + 2)
            d0 = pd0 - ps0
            d1 = pd1 - ps1
            d2 = pd2 - ps2
            r2 = jnp.maximum(d0 * d0 + d1 * d1 + d2 * d2, 1e-24)
            bits = plsc.bitcast(r2, jnp.int32)
            yb = jnp.int32(0x5F3759DF) - lax.shift_right_arithmetic(bits, 1)
            y = plsc.bitcast(yb, jnp.float32)
            y = y * (1.5 - 0.5 * r2 * y * y)
            y = y * (1.5 - 0.5 * r2 * y * y)
            y = y * (1.5 - 0.5 * r2 * y * y)
            ln = r2 * y
            t = jnp.minimum(ln * INV_H, NB - 0.5)
            jj = t.astype(jnp.int32)
            fr = t - jj.astype(jnp.float32)

            for d in range(EMBED):
                basev = plsc.load_gather(
                    lut_v, [jj, jnp.full((16,), d, jnp.int32)])
                slopev = plsc.load_gather(
                    lut_v, [jj, jnp.full((16,), EMBED + d, jnp.int32)])
                wv = basev + fr * slopev
                hv = gath(rowsA, d)
                plsc.store_scatter(
                    msg, [lane, jnp.full((16,), d, jnp.int32)], hv * wv)

    def issue_idx(ck, po):
        blk0 = wid * BLOCKS_PER_TILE + ck * BPC
        a = pltpu.async_copy(srcb.at[pl.ds(blk0, BPC)],
                             srci.at[pl.ds(po, BPC)], sem_i)
        b = pltpu.async_copy(dstb.at[pl.ds(blk0, BPC)],
                             dsti.at[pl.ds(po, BPC)], sem_i)
        return a, b

    def wait_idx(po):
        blk0 = 0
        pltpu.make_async_copy(srcb.at[pl.ds(blk0, BPC)],
                              srci.at[pl.ds(po, BPC)], sem_i).wait()
        pltpu.make_async_copy(dstb.at[pl.ds(blk0, BPC)],
                              dsti.at[pl.ds(po, BPC)], sem_i).wait()

    def issue_gathers(po, ro):
        for b in range(BPC):
            pltpu.async_copy(table.at[srci.at[po + b]],
                             rowsA.at[pl.ds(ro + b * BLK, BLK)], sem_g)
            pltpu.async_copy(pos16.at[dsti.at[po + b]],
                             posD.at[pl.ds(ro + b * BLK, BLK)], sem_g)

    def wait_gathers(po, ro):
        for b in range(BPC):
            pltpu.make_async_copy(table.at[srci.at[po + b]],
                                  rowsA.at[pl.ds(ro + b * BLK, BLK)],
                                  sem_g).wait()
            pltpu.make_async_copy(pos16.at[dsti.at[po + b]],
                                  posD.at[pl.ds(ro + b * BLK, BLK)],
                                  sem_g).wait()

    def issue_scatters(po, ro):
        for b in range(BPC):
            pltpu.async_copy(msg.at[pl.ds(ro + b * BLK, BLK)],
                             accum.at[dsti.at[po + b]], sem_s, add=True)

    def wait_scatters(po, ro):
        for b in range(BPC):
            pltpu.make_async_copy(msg.at[pl.ds(ro + b * BLK, BLK)],
                                  accum.at[dsti.at[po + b]],
                                  sem_s).wait()

    blk0 = wid * BLOCKS_PER_TILE
    pltpu.sync_copy(srcb.at[pl.ds(blk0, BPC)], srci.at[pl.ds(0, BPC)])
    pltpu.sync_copy(dstb.at[pl.ds(blk0, BPC)], dsti.at[pl.ds(0, BPC)])
    issue_gathers(0, 0)

    def chunk_loop(ck, carry):
        p = lax.rem(ck, 2)
        po = p * BPC
        ro = p * HALF
        q = 1 - p
        qo = q * BPC
        qro = q * HALF

        @pl.when(ck >= 1)
        def _():
            wait_scatters(qo, qro)

        @pl.when(ck < CHUNKS - 1)
        def _():
            issue_idx(ck + 1, qo)

        wait_gathers(po, ro)
        compute_chunk(ro)

        @pl.when(ck < CHUNKS - 1)
        def _():
            wait_idx(qo)
            issue_gathers(qo, qro)

        issue_scatters(po, ro)
        return carry

    lax.fori_loop(0, CHUNKS, chunk_loop, 0)
    lp = (CHUNKS - 1) % 2
    wait_scatters(lp * BPC, lp * HALF)
    plsc.subcore_barrier()

    pltpu.sync_copy(accum.at[pl.ds(s * 640, 640)],
                    part.at[c, pl.ds(s * 640, 640)])


def kernel(x, pos, edge_index, W_emb, b_emb, fc1_w, fc1_b, fc2_w, fc2_b):
    f32 = jnp.float32
    i32 = jnp.int32

    x_pad = jnp.pad(x.astype(f32), ((0, TABLE_ROWS - N_NODES), (0, 0)))
    pos_pad = jnp.pad(pos.astype(f32), ((0, TABLE_ROWS - N_NODES), (0, 0)))
    posc = jnp.pad(pos_pad, ((0, 0), (EMBED, 32 - EMBED - 3)))
    pos16 = jnp.pad(pos_pad, ((0, 0), (0, 13)))
    W_p = jnp.pad(W_emb.astype(f32), ((0, 0), (0, 32 - EMBED)))
    b_p = jnp.pad(b_emb.astype(f32), (0, 32 - EMBED)).reshape(1, 32)

    ei = edge_index.astype(i32)
    pad_e = E_PAD - N_EDGES
    src_p = jnp.concatenate(
        [ei[0], jnp.full((pad_e,), N_NODES, i32)]).reshape(E_PAD // BLK, BLK)
    dst_p = jnp.concatenate(
        [ei[1], jnp.full((pad_e,), N_NODES, i32)]).reshape(E_PAD // BLK, BLK)

    table, lut = pl.pallas_call(
        _tc_prepare,
        out_shape=[
            jax.ShapeDtypeStruct((TABLE_ROWS, 32), f32),
            jax.ShapeDtypeStruct((NB, 2 * EMBED), f32),
        ],
    )(x_pad, posc, W_p, b_p,
      fc1_w.astype(f32), fc1_b.astype(f32).reshape(1, EMB_NUMBER),
      fc2_w.astype(f32), fc2_b.astype(f32).reshape(1, EMBED))

    mesh = plsc.VectorSubcoreMesh(core_axis_name="c", subcore_axis_name="s")
    parts = pl.kernel(
        _sc_body,
        out_type=jax.ShapeDtypeStruct((NC, ACC_ROWS, 32), f32),
        mesh=mesh,
        compiler_params=pltpu.CompilerParams(
            needs_layout_passes=False, use_tc_tiling_on_sc=False),
        scratch_types=[
            pltpu.VMEM((NB, 2 * EMBED), f32),
            pltpu.VMEM((2 * BPC, BLK), i32),
            pltpu.VMEM((2 * BPC, BLK), i32),
            pltpu.VMEM((2 * BPC * BLK, 32), f32),
            pltpu.VMEM((2 * BPC * BLK, 16), f32),
            pltpu.VMEM((2 * BPC * BLK, 32), f32),
            pltpu.VMEM((16, 32), f32),
            pltpu.VMEM_SHARED((ACC_ROWS, 32), f32),
            pltpu.SemaphoreType.DMA,
            pltpu.SemaphoreType.DMA,
            pltpu.SemaphoreType.DMA,
        ],
    )(table, pos16, src_p, dst_p, lut)

    out = pl.pallas_call(
        _tc_combine,
        out_shape=jax.ShapeDtypeStruct((N_NODES, EMBED), f32),
    )(parts)
    return out
